# Initial kernel scaffold; baseline (speedup 1.0000x reference)
import jax
import jax.numpy as jnp
from jax.experimental import pallas as pl


def kernel(x, prototypes):
    raise NotImplementedError("write your pallas kernel here")


# speedup vs baseline: 1.4577x; 1.4577x over previous
import functools
import math

import jax
import jax.numpy as jnp
from jax.experimental import pallas as pl

_EPS = 1e-12


def _router_body(x_ref, p_ref, e_ref, w_ref, *, temp):
    xb = x_ref[...]
    p = p_ref[...]

    p_norm = jnp.sqrt(jnp.sum(p * p, axis=1, keepdims=True))
    p_n = p / jnp.maximum(p_norm, _EPS)

    sq = jnp.sum(xb * xb, axis=1, keepdims=True)
    x_norm = jnp.maximum(jnp.sqrt(sq), _EPS)
    xb_n = xb / x_norm

    s = jax.lax.dot_general(
        xb_n, p_n, (((1,), (1,)), ((), ())),
        preferred_element_type=jnp.float32)
    logits = s / temp

    E = logits.shape[1]
    idx = jax.lax.broadcasted_iota(jnp.int32, logits.shape, 1)
    m1 = jnp.max(logits, axis=1, keepdims=True)
    i1 = jnp.min(jnp.where(logits == m1, idx, E), axis=1, keepdims=True)
    masked = jnp.where(idx == i1, -jnp.inf, logits)
    m2 = jnp.max(masked, axis=1, keepdims=True)
    i2 = jnp.min(jnp.where(masked == m2, idx, E), axis=1, keepdims=True)

    e_ref[...] = jnp.concatenate([i1, i2], axis=1)
    z = jnp.exp(m2 - m1)
    p2 = z / (1.0 + z)
    w_ref[...] = jnp.concatenate([1.0 - p2, p2], axis=1)


@jax.jit
def kernel(x, prototypes):
    B, S, D = x.shape
    E = prototypes.shape[0]
    N = B * S
    temp = math.sqrt(D)
    T = 2048
    xf = x.reshape(N, D)

    experts, weights = pl.pallas_call(
        functools.partial(_router_body, temp=temp),
        grid=(N // T,),
        in_specs=[
            pl.BlockSpec((T, D), lambda i: (i, 0)),
            pl.BlockSpec((E, D), lambda i: (0, 0)),
        ],
        out_specs=[
            pl.BlockSpec((T, 2), lambda i: (i, 0)),
            pl.BlockSpec((T, 2), lambda i: (i, 0)),
        ],
        out_shape=[
            jax.ShapeDtypeStruct((N, 2), jnp.int32),
            jax.ShapeDtypeStruct((N, 2), jnp.float32),
        ],
    )(xf, prototypes)

    return experts.reshape(B, S, 2), weights.reshape(B, S, 2)


# speedup vs baseline: 1.4651x; 1.0051x over previous
import functools
import math

import jax
import jax.numpy as jnp
from jax.experimental import pallas as pl
from jax.experimental.pallas import tpu as pltpu

_EPS = 1e-12


def _router_body(x_ref, p_ref, e_ref, w_ref, pn_ref, *, temp):
    @pl.when(pl.program_id(0) == 0)
    def _normalize_prototypes():
        p = p_ref[...]
        p_norm = jnp.sqrt(jnp.sum(p * p, axis=1, keepdims=True))
        pn_ref[...] = p / jnp.maximum(p_norm, _EPS)

    xb = x_ref[...]
    p_n = pn_ref[...]

    sq = jnp.sum(xb * xb, axis=1, keepdims=True)
    x_norm = jnp.maximum(jnp.sqrt(sq), _EPS)
    xb_n = xb / x_norm

    s = jax.lax.dot_general(
        xb_n, p_n, (((1,), (1,)), ((), ())),
        preferred_element_type=jnp.float32)
    logits = s / temp

    E = logits.shape[1]
    idx = jax.lax.broadcasted_iota(jnp.int32, logits.shape, 1)
    m1 = jnp.max(logits, axis=1, keepdims=True)
    i1 = jnp.min(jnp.where(logits == m1, idx, E), axis=1, keepdims=True)
    masked = jnp.where(idx == i1, -jnp.inf, logits)
    m2 = jnp.max(masked, axis=1, keepdims=True)
    i2 = jnp.min(jnp.where(masked == m2, idx, E), axis=1, keepdims=True)

    e_ref[...] = jnp.concatenate([i1, i2], axis=1)
    z = jnp.exp(m2 - m1)
    p2 = z / (1.0 + z)
    w_ref[...] = jnp.concatenate([1.0 - p2, p2], axis=1)


@jax.jit
def kernel(x, prototypes):
    B, S, D = x.shape
    E = prototypes.shape[0]
    N = B * S
    temp = math.sqrt(D)
    T = 2048
    xf = x.reshape(N, D)

    experts, weights = pl.pallas_call(
        functools.partial(_router_body, temp=temp),
        grid=(N // T,),
        in_specs=[
            pl.BlockSpec((T, D), lambda i: (i, 0)),
            pl.BlockSpec((E, D), lambda i: (0, 0)),
        ],
        out_specs=[
            pl.BlockSpec((T, 2), lambda i: (i, 0)),
            pl.BlockSpec((T, 2), lambda i: (i, 0)),
        ],
        out_shape=[
            jax.ShapeDtypeStruct((N, 2), jnp.int32),
            jax.ShapeDtypeStruct((N, 2), jnp.float32),
        ],
        scratch_shapes=[pltpu.VMEM((E, D), jnp.float32)],
    )(xf, prototypes)

    return experts.reshape(B, S, 2), weights.reshape(B, S, 2)


# speedup vs baseline: 2.1130x; 1.4422x over previous
import functools
import math

import jax
import jax.numpy as jnp
from jax.experimental import pallas as pl
from jax.experimental.pallas import tpu as pltpu

_EPS = 1e-12


def _router_body(x_ref, p_ref, e_ref, w_ref, pn_ref, *, temp):
    @pl.when(pl.program_id(0) == 0)
    def _normalize_prototypes():
        p = p_ref[...]
        p_norm = jnp.sqrt(jnp.sum(p * p, axis=1, keepdims=True))
        pn_ref[...] = p / jnp.maximum(p_norm, _EPS)

    xb = x_ref[...]
    p_n = pn_ref[...]

    sq = jnp.sum(xb * xb, axis=1, keepdims=True)
    x_norm = jnp.maximum(jnp.sqrt(sq), _EPS)
    xb_n = xb / x_norm

    s = jax.lax.dot_general(
        xb_n, p_n, (((1,), (1,)), ((), ())),
        preferred_element_type=jnp.float32)
    logits = s.T / temp

    E = logits.shape[0]
    idx = jax.lax.broadcasted_iota(jnp.int32, logits.shape, 0)
    m1 = jnp.max(logits, axis=0, keepdims=True)
    i1 = jnp.min(jnp.where(logits == m1, idx, E), axis=0, keepdims=True)
    masked = jnp.where(idx == i1, -jnp.inf, logits)
    m2 = jnp.max(masked, axis=0, keepdims=True)
    i2 = jnp.min(jnp.where(masked == m2, idx, E), axis=0, keepdims=True)

    e_ref[...] = jnp.concatenate([i1, i2], axis=0)
    z = jnp.exp(m2 - m1)
    p2 = z / (1.0 + z)
    w_ref[...] = jnp.concatenate([1.0 - p2, p2], axis=0)


@jax.jit
def kernel(x, prototypes):
    B, S, D = x.shape
    E = prototypes.shape[0]
    N = B * S
    temp = math.sqrt(D)
    T = 2048
    xf = x.reshape(N, D)

    experts_t, weights_t = pl.pallas_call(
        functools.partial(_router_body, temp=temp),
        grid=(N // T,),
        in_specs=[
            pl.BlockSpec((T, D), lambda i: (i, 0)),
            pl.BlockSpec((E, D), lambda i: (0, 0)),
        ],
        out_specs=[
            pl.BlockSpec((2, T), lambda i: (0, i)),
            pl.BlockSpec((2, T), lambda i: (0, i)),
        ],
        out_shape=[
            jax.ShapeDtypeStruct((2, N), jnp.int32),
            jax.ShapeDtypeStruct((2, N), jnp.float32),
        ],
        scratch_shapes=[pltpu.VMEM((E, D), jnp.float32)],
    )(xf, prototypes)

    experts = experts_t.T.reshape(B, S, 2)
    weights = weights_t.T.reshape(B, S, 2)
    return experts, weights
